# baseline (speedup 1.0000x reference)
import functools

import jax
import jax.numpy as jnp
from jax import lax
from jax.experimental import pallas as pl
from jax.experimental.pallas import tpu as pltpu
from jax.experimental.pallas import tpu_sc as plsc

N_NODES = 10000
D = 128
NP = 10240
NW = 32
E_PER_W = 10240
EP = NW * E_PER_W
CHUNK = 128
NCHUNKS = E_PER_W // CHUNK
STRIPE = NP // 16

_mesh = plsc.VectorSubcoreMesh(
    core_axis_name="c", subcore_axis_name="s", num_cores=2, num_subcores=16
)

_DEG_CHUNK = 2560


@functools.partial(
    pl.kernel,
    out_type=jax.ShapeDtypeStruct((NW, NP), jnp.float32),
    mesh=_mesh,
    scratch_types=[
        pltpu.VMEM((NP,), jnp.float32),
        pltpu.VMEM((_DEG_CHUNK,), jnp.int32),
        pltpu.VMEM((_DEG_CHUNK,), jnp.float32),
    ],
    compiler_params=pltpu.CompilerParams(needs_layout_passes=False),
)
def _deg_kernel(dst_hbm, ew_hbm, out_hbm, deg_l, dst_b, ew_b):
    c = lax.axis_index("c")
    s = lax.axis_index("s")
    wid = s * 2 + c
    base = wid * E_PER_W
    zeros = jnp.zeros((16,), jnp.float32)

    def zero_body(i, _):
        deg_l[pl.ds(i * 16, 16)] = zeros
        return 0

    lax.fori_loop(0, NP // 16, zero_body, 0)

    def outer(j, _):
        off = base + j * _DEG_CHUNK
        pltpu.sync_copy(dst_hbm.at[pl.ds(off, _DEG_CHUNK)], dst_b)
        pltpu.sync_copy(ew_hbm.at[pl.ds(off, _DEG_CHUNK)], ew_b)

        def inner(g, _):
            dv = dst_b[pl.ds(g * 16, 16)]
            ev = ew_b[pl.ds(g * 16, 16)]
            plsc.addupdate_scatter(deg_l, [dv], ev)
            return 0

        lax.fori_loop(0, _DEG_CHUNK // 16, inner, 0)
        return 0

    lax.fori_loop(0, E_PER_W // _DEG_CHUNK, outer, 0)
    pltpu.sync_copy(deg_l, out_hbm.at[wid])


NBUF = 2
K0 = 40
K1 = 160 - K0


@functools.partial(
    pl.kernel,
    out_type=jax.ShapeDtypeStruct((2, NP, D), jnp.float32),
    mesh=_mesh,
    scratch_types=[
        pltpu.VMEM_SHARED((NP, D), jnp.float32),
        [pltpu.VMEM((3, CHUNK), jnp.int32) for _ in range(NBUF)],
        [pltpu.VMEM((CHUNK,), jnp.int32) for _ in range(NBUF)],
        [pltpu.VMEM((CHUNK, D), jnp.float32) for _ in range(NBUF)],
        [pltpu.SemaphoreType.DMA for _ in range(NBUF)],
        [pltpu.SemaphoreType.DMA for _ in range(NBUF)],
        [pltpu.SemaphoreType.DMA for _ in range(NBUF)],
    ],
    compiler_params=pltpu.CompilerParams(needs_layout_passes=False),
)
def _agg_kernel(meta_hbm, xw_hbm, out_hbm,
                agg_sh, bufs, dbufs, rows, semi, semg, sems):
    c = lax.axis_index("c")
    s = lax.axis_index("s")
    cbase = jnp.where(c == 0, s * K0, 16 * K0 + s * K1)
    niter = jnp.where(c == 0, K0 // NBUF, K1 // NBUF)
    zeros = jnp.zeros((16,), jnp.float32)

    def zrow(i, _):
        for f in range(D // 16):
            rows[0][i, pl.ds(f * 16, 16)] = zeros
        return 0

    lax.fori_loop(0, CHUNK, zrow, 0)

    def zstripe(k, _):
        pltpu.sync_copy(rows[0],
                        agg_sh.at[pl.ds(s * STRIPE + k * CHUNK, CHUNK)])
        return 0

    lax.fori_loop(0, STRIPE // CHUNK, zstripe, 0)
    plsc.subcore_barrier()

    for u in range(NBUF):
        pltpu.async_copy(meta_hbm.at[cbase + u], bufs[u], semi[u])

    def scale_grp(rbuf, mbuf, g, _):
        base_e = g * 16
        ev = plsc.bitcast(mbuf[2, pl.ds(base_e, 16)], jnp.float32)
        for e in range(16):
            w = ev[e]
            for f in range(D // 16):
                sl = pl.ds(f * 16, 16)
                rbuf[base_e + e, sl] = rbuf[base_e + e, sl] * w
        return 0

    def pipe_iter(k, _):
        for u in range(NBUF):
            j = k * NBUF + u
            pltpu.make_async_copy(meta_hbm.at[cbase + j], bufs[u],
                                  semi[u]).wait()

            @pl.when(k > 0)
            def _():
                pltpu.make_async_copy(rows[u], agg_sh.at[dbufs[u]],
                                      sems[u]).wait()

            for g in range(CHUNK // 16):
                dbufs[u][pl.ds(g * 16, 16)] = bufs[u][1, pl.ds(g * 16, 16)]
            pltpu.async_copy(xw_hbm.at[bufs[u].at[0]], rows[u], semg[u])

        for u in range(NBUF):
            j = k * NBUF + u
            pltpu.make_async_copy(xw_hbm.at[bufs[u].at[0]], rows[u],
                                  semg[u]).wait()
            lax.fori_loop(0, CHUNK // 16,
                          functools.partial(scale_grp, rows[u], bufs[u]), 0)
            pltpu.async_copy(rows[u], agg_sh.at[dbufs[u]], sems[u],
                             add=True)

            @pl.when(k < niter - 1)
            def _():
                pltpu.async_copy(meta_hbm.at[cbase + j + NBUF], bufs[u],
                                 semi[u])

        return 0

    lax.fori_loop(0, niter, pipe_iter, 0)
    for u in range(NBUF):
        pltpu.make_async_copy(rows[u], agg_sh.at[dbufs[u]], sems[u]).wait()
    plsc.subcore_barrier()

    def writeback(k, _):
        r0 = s * STRIPE + k * CHUNK
        pltpu.sync_copy(agg_sh.at[pl.ds(r0, CHUNK)], rows[0])
        pltpu.sync_copy(rows[0], out_hbm.at[c, pl.ds(r0, CHUNK)])
        return 0

    lax.fori_loop(0, STRIPE // CHUNK, writeback, 0)


def _mm_body(x_ref, w_ref, degp_ref, xw_ref, dis_ref):
    deg = jnp.sum(degp_ref[...], axis=0) + 2.0
    dis = jnp.where(deg > 0, lax.rsqrt(deg), 0.0)
    xw = jnp.dot(x_ref[...], w_ref[...], preferred_element_type=jnp.float32)
    xw_ref[...] = xw * dis[:, None]
    dis_ref[...] = dis[None, None]


_mm_call = pl.pallas_call(
    _mm_body,
    grid=(NP // 128,),
    in_specs=[
        pl.BlockSpec((128, D), lambda i: (i, 0)),
        pl.BlockSpec((D, D), lambda i: (0, 0)),
        pl.BlockSpec((NW, 128), lambda i: (0, i)),
    ],
    out_specs=[
        pl.BlockSpec((128, D), lambda i: (i, 0)),
        pl.BlockSpec((1, 1, 128), lambda i: (i, 0, 0)),
    ],
    out_shape=[
        jax.ShapeDtypeStruct((NP, D), jnp.float32),
        jax.ShapeDtypeStruct((NP // 128, 1, 128), jnp.float32),
    ],
)


def _fin_body(aggp_ref, xw_ref, dis_ref, b_ref, o_ref):
    agg = aggp_ref[0] + aggp_ref[1] + 2.0 * xw_ref[...]
    o_ref[...] = jnp.tanh(dis_ref[0, 0][:, None] * agg + b_ref[0][None, :])


_fin_call = pl.pallas_call(
    _fin_body,
    grid=(NP // 128,),
    in_specs=[
        pl.BlockSpec((2, 128, D), lambda i: (0, i, 0)),
        pl.BlockSpec((128, D), lambda i: (i, 0)),
        pl.BlockSpec((1, 1, 128), lambda i: (i, 0, 0)),
        pl.BlockSpec((1, D), lambda i: (0, 0)),
    ],
    out_specs=pl.BlockSpec((128, D), lambda i: (i, 0)),
    out_shape=jax.ShapeDtypeStruct((NP, D), jnp.float32),
)


def kernel(x, edge_index, edge_weight, W, b):
    src = edge_index[0].astype(jnp.int32)
    dst = edge_index[1].astype(jnp.int32)
    ew = edge_weight.astype(jnp.float32)
    pad_e = EP - src.shape[0]
    src_p = jnp.pad(src, (0, pad_e))
    dst_p = jnp.pad(dst, (0, pad_e))
    ew_p = jnp.pad(ew, (0, pad_e))
    x_p = jnp.pad(x, ((0, NP - x.shape[0]), (0, 0)))
    meta = jnp.concatenate(
        [
            src_p.reshape(-1, 1, CHUNK),
            dst_p.reshape(-1, 1, CHUNK),
            lax.bitcast_convert_type(ew_p, jnp.int32).reshape(-1, 1, CHUNK),
        ],
        axis=1,
    )

    degp = _deg_kernel(dst_p, ew_p)
    xw_s, dis = _mm_call(x_p, W, degp)
    aggp = _agg_kernel(meta, xw_s)
    out = _fin_call(aggp, xw_s, dis, b.reshape(1, D))
    return out[:N_NODES]


# speedup vs baseline: 1.1829x; 1.1829x over previous
import functools

import jax
import jax.numpy as jnp
from jax import lax
from jax.experimental import pallas as pl
from jax.experimental.pallas import tpu as pltpu
from jax.experimental.pallas import tpu_sc as plsc

N_NODES = 10000
D = 128
NP = 10240
NW = 32
E_PER_W = 10240
EP = NW * E_PER_W
CHUNK = 128
NCHUNKS = E_PER_W // CHUNK
STRIPE = NP // 16

_mesh = plsc.VectorSubcoreMesh(
    core_axis_name="c", subcore_axis_name="s", num_cores=2, num_subcores=16
)

_DEG_CHUNK = 2560


@functools.partial(
    pl.kernel,
    out_type=jax.ShapeDtypeStruct((NW, NP), jnp.float32),
    mesh=_mesh,
    scratch_types=[
        pltpu.VMEM((NP,), jnp.float32),
        pltpu.VMEM((_DEG_CHUNK,), jnp.int32),
        pltpu.VMEM((_DEG_CHUNK,), jnp.float32),
    ],
    compiler_params=pltpu.CompilerParams(needs_layout_passes=False),
)
def _deg_kernel(dst_hbm, ew_hbm, out_hbm, deg_l, dst_b, ew_b):
    c = lax.axis_index("c")
    s = lax.axis_index("s")
    wid = s * 2 + c
    base = wid * E_PER_W
    zeros = jnp.zeros((16,), jnp.float32)

    def zero_body(i, _):
        deg_l[pl.ds(i * 16, 16)] = zeros
        return 0

    lax.fori_loop(0, NP // 16, zero_body, 0)

    def outer(j, _):
        off = base + j * _DEG_CHUNK
        pltpu.sync_copy(dst_hbm.at[pl.ds(off, _DEG_CHUNK)], dst_b)
        pltpu.sync_copy(ew_hbm.at[pl.ds(off, _DEG_CHUNK)], ew_b)

        def inner(g, _):
            dv = dst_b[pl.ds(g * 16, 16)]
            ev = ew_b[pl.ds(g * 16, 16)]
            plsc.addupdate_scatter(deg_l, [dv], ev)
            return 0

        lax.fori_loop(0, _DEG_CHUNK // 16, inner, 0)
        return 0

    lax.fori_loop(0, E_PER_W // _DEG_CHUNK, outer, 0)
    pltpu.sync_copy(deg_l, out_hbm.at[wid])


NBUF = 2
K0 = 120
K1 = 160 - K0


@functools.partial(
    pl.kernel,
    out_type=jax.ShapeDtypeStruct((2, NP, D), jnp.float32),
    mesh=_mesh,
    scratch_types=[
        pltpu.VMEM_SHARED((NP, D), jnp.float32),
        [pltpu.VMEM((3, CHUNK), jnp.int32) for _ in range(NBUF)],
        [pltpu.VMEM((CHUNK,), jnp.int32) for _ in range(NBUF)],
        [pltpu.VMEM((CHUNK, D), jnp.float32) for _ in range(NBUF)],
        [pltpu.SemaphoreType.DMA for _ in range(NBUF)],
        [pltpu.SemaphoreType.DMA for _ in range(NBUF)],
        [pltpu.SemaphoreType.DMA for _ in range(NBUF)],
    ],
    compiler_params=pltpu.CompilerParams(needs_layout_passes=False),
)
def _agg_kernel(meta_hbm, xw_hbm, out_hbm,
                agg_sh, bufs, dbufs, rows, semi, semg, sems):
    c = lax.axis_index("c")
    s = lax.axis_index("s")
    cbase = jnp.where(c == 0, s * K0, 16 * K0 + s * K1)
    niter = jnp.where(c == 0, K0 // NBUF, K1 // NBUF)
    zeros = jnp.zeros((16,), jnp.float32)

    def zrow(i, _):
        for f in range(D // 16):
            rows[0][i, pl.ds(f * 16, 16)] = zeros
        return 0

    lax.fori_loop(0, CHUNK, zrow, 0)

    def zstripe(k, _):
        pltpu.sync_copy(rows[0],
                        agg_sh.at[pl.ds(s * STRIPE + k * CHUNK, CHUNK)])
        return 0

    lax.fori_loop(0, STRIPE // CHUNK, zstripe, 0)
    plsc.subcore_barrier()

    for u in range(NBUF):
        pltpu.async_copy(meta_hbm.at[cbase + u], bufs[u], semi[u])

    def scale_grp(rbuf, mbuf, g, _):
        base_e = g * 16
        ev = plsc.bitcast(mbuf[2, pl.ds(base_e, 16)], jnp.float32)
        for e in range(16):
            w = ev[e]
            for f in range(D // 16):
                sl = pl.ds(f * 16, 16)
                rbuf[base_e + e, sl] = rbuf[base_e + e, sl] * w
        return 0

    def pipe_iter(k, _):
        for u in range(NBUF):
            j = k * NBUF + u
            pltpu.make_async_copy(meta_hbm.at[cbase + j], bufs[u],
                                  semi[u]).wait()

            @pl.when(k > 0)
            def _():
                pltpu.make_async_copy(rows[u], agg_sh.at[dbufs[u]],
                                      sems[u]).wait()

            for g in range(CHUNK // 16):
                dbufs[u][pl.ds(g * 16, 16)] = bufs[u][1, pl.ds(g * 16, 16)]
            pltpu.async_copy(xw_hbm.at[bufs[u].at[0]], rows[u], semg[u])

        for u in range(NBUF):
            j = k * NBUF + u
            pltpu.make_async_copy(xw_hbm.at[bufs[u].at[0]], rows[u],
                                  semg[u]).wait()
            lax.fori_loop(0, CHUNK // 16,
                          functools.partial(scale_grp, rows[u], bufs[u]), 0)
            pltpu.async_copy(rows[u], agg_sh.at[dbufs[u]], sems[u],
                             add=True)

            @pl.when(k < niter - 1)
            def _():
                pltpu.async_copy(meta_hbm.at[cbase + j + NBUF], bufs[u],
                                 semi[u])

        return 0

    lax.fori_loop(0, niter, pipe_iter, 0)
    for u in range(NBUF):
        pltpu.make_async_copy(rows[u], agg_sh.at[dbufs[u]], sems[u]).wait()
    plsc.subcore_barrier()

    def writeback(k, _):
        r0 = s * STRIPE + k * CHUNK
        pltpu.sync_copy(agg_sh.at[pl.ds(r0, CHUNK)], rows[0])
        pltpu.sync_copy(rows[0], out_hbm.at[c, pl.ds(r0, CHUNK)])
        return 0

    lax.fori_loop(0, STRIPE // CHUNK, writeback, 0)


def _mm_body(x_ref, w_ref, degp_ref, xw_ref, dis_ref):
    deg = jnp.sum(degp_ref[...], axis=0) + 2.0
    dis = jnp.where(deg > 0, lax.rsqrt(deg), 0.0)
    xw = jnp.dot(x_ref[...], w_ref[...], preferred_element_type=jnp.float32)
    xw_ref[...] = xw * dis[:, None]
    dis_ref[...] = dis[None, None]


_mm_call = pl.pallas_call(
    _mm_body,
    grid=(NP // 128,),
    in_specs=[
        pl.BlockSpec((128, D), lambda i: (i, 0)),
        pl.BlockSpec((D, D), lambda i: (0, 0)),
        pl.BlockSpec((NW, 128), lambda i: (0, i)),
    ],
    out_specs=[
        pl.BlockSpec((128, D), lambda i: (i, 0)),
        pl.BlockSpec((1, 1, 128), lambda i: (i, 0, 0)),
    ],
    out_shape=[
        jax.ShapeDtypeStruct((NP, D), jnp.float32),
        jax.ShapeDtypeStruct((NP // 128, 1, 128), jnp.float32),
    ],
)


def _fin_body(aggp_ref, xw_ref, dis_ref, b_ref, o_ref):
    agg = aggp_ref[0] + aggp_ref[1] + 2.0 * xw_ref[...]
    o_ref[...] = jnp.tanh(dis_ref[0, 0][:, None] * agg + b_ref[0][None, :])


_fin_call = pl.pallas_call(
    _fin_body,
    grid=(NP // 128,),
    in_specs=[
        pl.BlockSpec((2, 128, D), lambda i: (0, i, 0)),
        pl.BlockSpec((128, D), lambda i: (i, 0)),
        pl.BlockSpec((1, 1, 128), lambda i: (i, 0, 0)),
        pl.BlockSpec((1, D), lambda i: (0, 0)),
    ],
    out_specs=pl.BlockSpec((128, D), lambda i: (i, 0)),
    out_shape=jax.ShapeDtypeStruct((NP, D), jnp.float32),
)


def kernel(x, edge_index, edge_weight, W, b):
    src = edge_index[0].astype(jnp.int32)
    dst = edge_index[1].astype(jnp.int32)
    ew = edge_weight.astype(jnp.float32)
    pad_e = EP - src.shape[0]
    src_p = jnp.pad(src, (0, pad_e))
    dst_p = jnp.pad(dst, (0, pad_e))
    ew_p = jnp.pad(ew, (0, pad_e))
    x_p = jnp.pad(x, ((0, NP - x.shape[0]), (0, 0)))
    meta = jnp.concatenate(
        [
            src_p.reshape(-1, 1, CHUNK),
            dst_p.reshape(-1, 1, CHUNK),
            lax.bitcast_convert_type(ew_p, jnp.int32).reshape(-1, 1, CHUNK),
        ],
        axis=1,
    )

    degp = _deg_kernel(dst_p, ew_p)
    xw_s, dis = _mm_call(x_p, W, degp)
    aggp = _agg_kernel(meta, xw_s)
    out = _fin_call(aggp, xw_s, dis, b.reshape(1, D))
    return out[:N_NODES]


# speedup vs baseline: 1.2073x; 1.0206x over previous
import functools

import jax
import jax.numpy as jnp
from jax import lax
from jax.experimental import pallas as pl
from jax.experimental.pallas import tpu as pltpu
from jax.experimental.pallas import tpu_sc as plsc

N_NODES = 10000
D = 128
NP = 10240
NW = 32
E_PER_W = 10240
EP = NW * E_PER_W
CHUNK = 128
NCHUNKS = E_PER_W // CHUNK
STRIPE = NP // 16

_mesh = plsc.VectorSubcoreMesh(
    core_axis_name="c", subcore_axis_name="s", num_cores=2, num_subcores=16
)

_DEG_CHUNK = 2560


@functools.partial(
    pl.kernel,
    out_type=jax.ShapeDtypeStruct((NW, NP), jnp.float32),
    mesh=_mesh,
    scratch_types=[
        pltpu.VMEM((NP,), jnp.float32),
        pltpu.VMEM((_DEG_CHUNK,), jnp.int32),
        pltpu.VMEM((_DEG_CHUNK,), jnp.float32),
    ],
    compiler_params=pltpu.CompilerParams(needs_layout_passes=False),
)
def _deg_kernel(dst_hbm, ew_hbm, out_hbm, deg_l, dst_b, ew_b):
    c = lax.axis_index("c")
    s = lax.axis_index("s")
    wid = s * 2 + c
    base = wid * E_PER_W
    zeros = jnp.zeros((16,), jnp.float32)

    def zero_body(i, _):
        deg_l[pl.ds(i * 16, 16)] = zeros
        return 0

    lax.fori_loop(0, NP // 16, zero_body, 0)

    def outer(j, _):
        off = base + j * _DEG_CHUNK
        pltpu.sync_copy(dst_hbm.at[pl.ds(off, _DEG_CHUNK)], dst_b)
        pltpu.sync_copy(ew_hbm.at[pl.ds(off, _DEG_CHUNK)], ew_b)

        def inner(g, _):
            dv = dst_b[pl.ds(g * 16, 16)]
            ev = ew_b[pl.ds(g * 16, 16)]
            plsc.addupdate_scatter(deg_l, [dv], ev)
            return 0

        lax.fori_loop(0, _DEG_CHUNK // 16, inner, 0)
        return 0

    lax.fori_loop(0, E_PER_W // _DEG_CHUNK, outer, 0)
    pltpu.sync_copy(deg_l, out_hbm.at[wid])


NBUF = 4
NITER = NCHUNKS // NBUF
DH = D // 2


@functools.partial(
    pl.kernel,
    out_type=jax.ShapeDtypeStruct((2, 2, NP, DH), jnp.float32),
    mesh=_mesh,
    scratch_types=[
        pltpu.VMEM_SHARED((NP, DH), jnp.float32),
        pltpu.VMEM_SHARED((NP, DH), jnp.float32),
        pltpu.VMEM((CHUNK, DH), jnp.float32),
        [pltpu.VMEM((3, CHUNK), jnp.int32) for _ in range(NBUF)],
        [pltpu.VMEM((CHUNK,), jnp.int32) for _ in range(NBUF)],
        [pltpu.VMEM((CHUNK, DH), jnp.float32) for _ in range(NBUF)],
        [pltpu.SemaphoreType.DMA for _ in range(NBUF)],
        [pltpu.SemaphoreType.DMA for _ in range(NBUF)],
        [pltpu.SemaphoreType.DMA for _ in range(NBUF)],
    ],
    compiler_params=pltpu.CompilerParams(needs_layout_passes=False,
                                         use_tc_tiling_on_sc=False),
)
def _agg_kernel(meta_hbm, xwl_hbm, xwh_hbm, out_hbm,
                agg_sh, xw_sh, zbuf, bufs, dbufs, rows, semi, semg, sems):
    c = lax.axis_index("c")
    s = lax.axis_index("s")
    wid = s * 2 + c
    cbase = wid * NCHUNKS
    zeros = jnp.zeros((16,), jnp.float32)

    def zrow(i, _):
        for f in range(DH // 16):
            zbuf[i, pl.ds(f * 16, 16)] = zeros
        return 0

    lax.fori_loop(0, CHUNK, zrow, 0)

    def scale_grp(rbuf, mbuf, g, _):
        base_e = g * 16
        ev = plsc.bitcast(mbuf[2, pl.ds(base_e, 16)], jnp.float32)
        for e in range(16):
            w = ev[e]
            for f in range(DH // 16):
                sl = pl.ds(f * 16, 16)
                rbuf[base_e + e, sl] = rbuf[base_e + e, sl] * w
        return 0

    for h, xw_half in enumerate((xwl_hbm, xwh_hbm)):
        pltpu.sync_copy(xw_half.at[pl.ds(s * STRIPE, STRIPE)],
                        xw_sh.at[pl.ds(s * STRIPE, STRIPE)])

        def zstripe(k, _):
            pltpu.sync_copy(zbuf,
                            agg_sh.at[pl.ds(s * STRIPE + k * CHUNK, CHUNK)])
            return 0

        lax.fori_loop(0, STRIPE // CHUNK, zstripe, 0)
        plsc.subcore_barrier()

        for u in range(NBUF):
            pltpu.async_copy(meta_hbm.at[cbase + u], bufs[u], semi[u])

        def pipe_iter(k, _):
            for u in range(NBUF):
                j = k * NBUF + u
                pltpu.make_async_copy(meta_hbm.at[cbase + j], bufs[u],
                                      semi[u]).wait()

                @pl.when(k > 0)
                def _():
                    pltpu.make_async_copy(rows[u], agg_sh.at[dbufs[u]],
                                          sems[u]).wait()

                for g in range(CHUNK // 16):
                    dbufs[u][pl.ds(g * 16, 16)] = \
                        bufs[u][1, pl.ds(g * 16, 16)]
                pltpu.async_copy(xw_sh.at[bufs[u].at[0]], rows[u], semg[u])

            for u in range(NBUF):
                j = k * NBUF + u
                pltpu.make_async_copy(xw_sh.at[bufs[u].at[0]], rows[u],
                                      semg[u]).wait()
                lax.fori_loop(0, CHUNK // 16,
                              functools.partial(scale_grp, rows[u], bufs[u]),
                              0)
                pltpu.async_copy(rows[u], agg_sh.at[dbufs[u]], sems[u],
                                 add=True)

                @pl.when(k < NITER - 1)
                def _():
                    pltpu.async_copy(meta_hbm.at[cbase + j + NBUF], bufs[u],
                                     semi[u])

            return 0

        lax.fori_loop(0, NITER, pipe_iter, 0)
        for u in range(NBUF):
            pltpu.make_async_copy(rows[u], agg_sh.at[dbufs[u]],
                                  sems[u]).wait()
        plsc.subcore_barrier()

        def writeback(k, _):
            r0 = s * STRIPE + k * CHUNK
            pltpu.sync_copy(agg_sh.at[pl.ds(r0, CHUNK)], rows[0])
            pltpu.sync_copy(rows[0], out_hbm.at[c, h, pl.ds(r0, CHUNK)])
            return 0

        lax.fori_loop(0, STRIPE // CHUNK, writeback, 0)


def _mm_body(x_ref, w_ref, degp_ref, xwl_ref, xwh_ref, dis_ref):
    deg = jnp.sum(degp_ref[...], axis=0) + 2.0
    dis = jnp.where(deg > 0, lax.rsqrt(deg), 0.0)
    xw = jnp.dot(x_ref[...], w_ref[...], preferred_element_type=jnp.float32)
    xw = xw * dis[:, None]
    xwl_ref[...] = xw[:, :DH]
    xwh_ref[...] = xw[:, DH:]
    dis_ref[...] = dis[None, None]


_mm_call = pl.pallas_call(
    _mm_body,
    grid=(NP // 128,),
    in_specs=[
        pl.BlockSpec((128, D), lambda i: (i, 0)),
        pl.BlockSpec((D, D), lambda i: (0, 0)),
        pl.BlockSpec((NW, 128), lambda i: (0, i)),
    ],
    out_specs=[
        pl.BlockSpec((128, DH), lambda i: (i, 0)),
        pl.BlockSpec((128, DH), lambda i: (i, 0)),
        pl.BlockSpec((1, 1, 128), lambda i: (i, 0, 0)),
    ],
    out_shape=[
        jax.ShapeDtypeStruct((NP, DH), jnp.float32),
        jax.ShapeDtypeStruct((NP, DH), jnp.float32),
        jax.ShapeDtypeStruct((NP // 128, 1, 128), jnp.float32),
    ],
)


def _fin_body(aggp_ref, xwl_ref, xwh_ref, dis_ref, b_ref, o_ref):
    a = aggp_ref[...]
    agg = jnp.concatenate(
        [a[0, 0] + a[1, 0] + 2.0 * xwl_ref[...],
         a[0, 1] + a[1, 1] + 2.0 * xwh_ref[...]],
        axis=-1,
    )
    o_ref[...] = jnp.tanh(dis_ref[0, 0][:, None] * agg + b_ref[0][None, :])


_fin_call = pl.pallas_call(
    _fin_body,
    grid=(NP // 128,),
    in_specs=[
        pl.BlockSpec((2, 2, 128, DH), lambda i: (0, 0, i, 0)),
        pl.BlockSpec((128, DH), lambda i: (i, 0)),
        pl.BlockSpec((128, DH), lambda i: (i, 0)),
        pl.BlockSpec((1, 1, 128), lambda i: (i, 0, 0)),
        pl.BlockSpec((1, D), lambda i: (0, 0)),
    ],
    out_specs=pl.BlockSpec((128, D), lambda i: (i, 0)),
    out_shape=jax.ShapeDtypeStruct((NP, D), jnp.float32),
)


def kernel(x, edge_index, edge_weight, W, b):
    src = edge_index[0].astype(jnp.int32)
    dst = edge_index[1].astype(jnp.int32)
    ew = edge_weight.astype(jnp.float32)
    pad_e = EP - src.shape[0]
    src_p = jnp.pad(src, (0, pad_e))
    dst_p = jnp.pad(dst, (0, pad_e))
    ew_p = jnp.pad(ew, (0, pad_e))
    x_p = jnp.pad(x, ((0, NP - x.shape[0]), (0, 0)))
    meta = jnp.concatenate(
        [
            src_p.reshape(-1, 1, CHUNK),
            dst_p.reshape(-1, 1, CHUNK),
            lax.bitcast_convert_type(ew_p, jnp.int32).reshape(-1, 1, CHUNK),
        ],
        axis=1,
    )

    degp = _deg_kernel(dst_p, ew_p)
    xwl, xwh, dis = _mm_call(x_p, W, degp)
    aggp = _agg_kernel(meta, xwl, xwh)
    out = _fin_call(aggp, xwl, xwh, dis, b.reshape(1, D))
    return out[:N_NODES]
